# Initial kernel scaffold; baseline (speedup 1.0000x reference)
import jax
import jax.numpy as jnp
from jax.experimental import pallas as pl


def kernel(samples):
    raise NotImplementedError("write your pallas kernel here")


# speedup vs baseline: 1.7969x; 1.7969x over previous
import functools

import jax
import jax.numpy as jnp
from jax import lax
from jax.experimental import pallas as pl
from jax.experimental.pallas import tpu as pltpu
from jax.experimental.pallas import tpu_sc as plsc

N = 16384
D = 256
L = 16
NW = 32
COLS_PER_W = D // NW
C = N // L
NBINS = 256
INV = 1.0 / (N + 1)

_mesh = plsc.VectorSubcoreMesh(core_axis_name="c", subcore_axis_name="s")


@functools.partial(
    pl.kernel,
    out_type=jax.ShapeDtypeStruct((1, D, N), jnp.float32),
    mesh=_mesh,
    scratch_types=[
        pltpu.VMEM((N,), jnp.float32),
        pltpu.VMEM((N,), jnp.int32),
        pltpu.VMEM((N,), jnp.int32),
        pltpu.VMEM((N,), jnp.int32),
        pltpu.VMEM((N,), jnp.int32),
        pltpu.VMEM((N,), jnp.float32),
        pltpu.VMEM((NBINS * L,), jnp.int32),
    ],
    compiler_params=pltpu.CompilerParams(needs_layout_passes=False),
)
def _rank_all_columns(x_hbm, out_hbm, inbuf, key_a, key_b, pay_a, pay_b,
                      outbuf, hist):
    wid = lax.axis_index("s") * 2 + lax.axis_index("c")
    lanes = lax.iota(jnp.int32, L)
    lane_base = lanes * C
    ones = jnp.ones((L,), jnp.int32)
    zeros = jnp.zeros((L,), jnp.int32)

    def to_key(xf):
        xi = lax.bitcast_convert_type(xf + 0.0, jnp.int32)
        m = lax.shift_right_arithmetic(xi, 31)
        return lax.bitwise_xor(xi, lax.bitwise_or(m, jnp.int32(-(2**31))))

    def slot_of(key, sh):
        d = lax.bitwise_and(lax.shift_right_logical(key, sh), jnp.int32(0xFF))
        return lax.shift_left(d, jnp.int32(4)) + lanes

    def zero_hist():
        def zf(i, carry):
            hist[pl.ds(i * L, L)] = zeros
            return carry
        lax.fori_loop(0, NBINS, zf, 0)

    def scan_hist():
        def sf(i, carry):
            v = hist[pl.ds(i * L, L)]
            inc = plsc.cumsum(v)
            hist[pl.ds(i * L, L)] = inc - v + carry
            return carry + jnp.sum(v)
        lax.fori_loop(0, NBINS, sf, jnp.int32(0))

    def radix_pass(sh, get_key, get_pay, emit):
        zero_hist()

        def histf(t, carry):
            k = get_key(t)
            plsc.addupdate_scatter(hist, [slot_of(k, sh)], ones)
            return carry
        lax.fori_loop(0, C, histf, 0)

        scan_hist()

        def permf(t, carry):
            k = get_key(t)
            s = slot_of(k, sh)
            p = plsc.load_gather(hist, [s])
            plsc.addupdate_scatter(hist, [s], ones)
            emit(p, k, get_pay(t))
            return carry
        lax.fori_loop(0, C, permf, 0)

    def do_col(ci, carry):
        col = wid * COLS_PER_W + ci
        pltpu.sync_copy(x_hbm.at[col], inbuf)

        def k_in(t):
            return to_key(plsc.load_gather(inbuf, [lane_base + t]))

        def p_iota(t):
            return lane_base + t

        def emit_ab(p, k, pv):
            plsc.store_scatter(key_a, [p], k)
            plsc.store_scatter(pay_a, [p], pv)

        radix_pass(0, k_in, p_iota, emit_ab)

        def k_a(t):
            return plsc.load_gather(key_a, [lane_base + t])

        def p_a(t):
            return plsc.load_gather(pay_a, [lane_base + t])

        def emit_ba(p, k, pv):
            plsc.store_scatter(key_b, [p], k)
            plsc.store_scatter(pay_b, [p], pv)

        radix_pass(8, k_a, p_a, emit_ba)

        def k_b(t):
            return plsc.load_gather(key_b, [lane_base + t])

        def p_b(t):
            return plsc.load_gather(pay_b, [lane_base + t])

        radix_pass(16, k_b, p_b, emit_ab)

        def emit_out(p, k, pv):
            val = lax.convert_element_type(p + 1, jnp.float32) * INV
            plsc.store_scatter(outbuf, [pv], val)

        radix_pass(24, k_a, p_a, emit_out)

        pltpu.sync_copy(outbuf, out_hbm.at[0, col])
        return carry

    lax.fori_loop(0, COLS_PER_W, do_col, 0)


def kernel(samples):
    xt = jnp.transpose(samples)
    return _rank_all_columns(xt)
